# Initial kernel scaffold; baseline (speedup 1.0000x reference)
import jax
import jax.numpy as jnp
from jax.experimental import pallas as pl


def kernel(token_embeddings, uzman_embeddings, onbellek_durumu):
    raise NotImplementedError("write your pallas kernel here")


# speedup vs baseline: 3.5376x; 3.5376x over previous
import jax
import jax.numpy as jnp
from jax.experimental import pallas as pl
from jax.experimental.pallas import tpu as pltpu

E = 64
L = 8
K = 2
NEG = -jnp.inf


def _router_kernel(x_ref, w_ref, onb_ref, idx_ref, wgt_ref):
    x = x_ref[...]
    w = w_ref[...]
    dims = (((1,), (1,)), ((), ()))
    sims = jax.lax.dot_general(x, w, dims,
                               preferred_element_type=jnp.float32)

    inv_tn = 1.0 / (jnp.sqrt(jnp.sum(x * x, axis=1, keepdims=True)) + 1e-8)
    inv_en = 1.0 / (jnp.sqrt(jnp.sum(w * w, axis=1, keepdims=True)) + 1e-8)
    cos = jax.lax.dot_general(x * inv_tn, w * inv_en, dims,
                              preferred_element_type=jnp.float32)
    total = cos + 0.1 * onb_ref[0, :][None, :]

    s = sims
    for _ in range(L):
        m = jnp.max(s, axis=1, keepdims=True)
        s = jnp.where(s == m, NEG, s)

    t = jnp.where(s == NEG, total, NEG)
    iota = jax.lax.broadcasted_iota(jnp.int32, t.shape, 1)
    v1 = jnp.max(t, axis=1, keepdims=True)
    i1 = jnp.min(jnp.where(t == v1, iota, E), axis=1, keepdims=True)
    t2 = jnp.where(iota == i1, NEG, t)
    v2 = jnp.max(t2, axis=1, keepdims=True)
    i2 = jnp.min(jnp.where(t2 == v2, iota, E), axis=1, keepdims=True)

    w1 = 1.0 / (1.0 + jnp.exp(v2 - v1))
    idx_ref[...] = jnp.concatenate([i1, i2], axis=1).astype(jnp.int32)
    wgt_ref[...] = jnp.concatenate([w1, 1.0 - w1], axis=1)


def kernel(token_embeddings, uzman_embeddings, onbellek_durumu):
    B, S, H = token_embeddings.shape
    N = B * S
    T = 1024
    x = token_embeddings.reshape(N, H)
    onb = onbellek_durumu.reshape(1, E)

    idx, wgt = pl.pallas_call(
        _router_kernel,
        grid=(N // T,),
        in_specs=[
            pl.BlockSpec((T, H), lambda i: (i, 0)),
            pl.BlockSpec((E, H), lambda i: (0, 0)),
            pl.BlockSpec((1, E), lambda i: (0, 0)),
        ],
        out_specs=[
            pl.BlockSpec((T, K), lambda i: (i, 0)),
            pl.BlockSpec((T, K), lambda i: (i, 0)),
        ],
        out_shape=[
            jax.ShapeDtypeStruct((N, K), jnp.int32),
            jax.ShapeDtypeStruct((N, K), jnp.float32),
        ],
        compiler_params=pltpu.CompilerParams(
            dimension_semantics=("arbitrary",),
        ),
    )(x, uzman_embeddings, onb)

    return idx.reshape(B, S, K), wgt.reshape(B, S, K)


# speedup vs baseline: 5.9970x; 1.6952x over previous
import jax
import jax.numpy as jnp
from jax.experimental import pallas as pl
from jax.experimental.pallas import tpu as pltpu

E = 64
L = 8
K = 2
NEG = -jnp.inf
OUTR = 8


def _router_kernel(x_ref, w_ref, onb_ref, idx_ref, wgt_ref):
    x = x_ref[...]
    w = w_ref[...]
    dims = (((1,), (1,)), ((), ()))
    sims = jax.lax.dot_general(w, x, dims,
                               preferred_element_type=jnp.float32)

    inv_tn = 1.0 / (jnp.sqrt(jnp.sum(x * x, axis=1, keepdims=True)) + 1e-8)
    inv_en = 1.0 / (jnp.sqrt(jnp.sum(w * w, axis=1, keepdims=True)) + 1e-8)
    cos = jax.lax.dot_general(w * inv_en, x * inv_tn, dims,
                              preferred_element_type=jnp.float32)
    total = cos + 0.1 * onb_ref[...]

    s = sims
    for _ in range(L):
        m = jnp.max(s, axis=0, keepdims=True)
        s = jnp.where(s == m, NEG, s)

    t = jnp.where(s == NEG, total, NEG)
    iota = jax.lax.broadcasted_iota(jnp.int32, t.shape, 0)
    v1 = jnp.max(t, axis=0, keepdims=True)
    i1 = jnp.min(jnp.where(t == v1, iota, E), axis=0, keepdims=True)
    t2 = jnp.where(iota == i1, NEG, t)
    v2 = jnp.max(t2, axis=0, keepdims=True)
    i2 = jnp.min(jnp.where(t2 == v2, iota, E), axis=0, keepdims=True)

    w1 = 1.0 / (1.0 + jnp.exp(v2 - v1))
    T = x.shape[0]
    zi = jnp.zeros((OUTR - K, T), jnp.int32)
    zf = jnp.zeros((OUTR - K, T), jnp.float32)
    idx_ref[...] = jnp.concatenate([i1, i2, zi], axis=0)
    wgt_ref[...] = jnp.concatenate([w1, 1.0 - w1, zf], axis=0)


def kernel(token_embeddings, uzman_embeddings, onbellek_durumu):
    B, S, H = token_embeddings.shape
    N = B * S
    T = 1024
    x = token_embeddings.reshape(N, H)
    onb = onbellek_durumu.reshape(E, 1)

    idx, wgt = pl.pallas_call(
        _router_kernel,
        grid=(N // T,),
        in_specs=[
            pl.BlockSpec((T, H), lambda i: (i, 0)),
            pl.BlockSpec((E, H), lambda i: (0, 0)),
            pl.BlockSpec((E, 1), lambda i: (0, 0)),
        ],
        out_specs=[
            pl.BlockSpec((OUTR, T), lambda i: (0, i)),
            pl.BlockSpec((OUTR, T), lambda i: (0, i)),
        ],
        out_shape=[
            jax.ShapeDtypeStruct((OUTR, N), jnp.int32),
            jax.ShapeDtypeStruct((OUTR, N), jnp.float32),
        ],
        compiler_params=pltpu.CompilerParams(
            dimension_semantics=("arbitrary",),
        ),
    )(x, uzman_embeddings, onb)

    idx = idx[:K].T.reshape(B, S, K)
    wgt = wgt[:K].T.reshape(B, S, K)
    return idx, wgt


# speedup vs baseline: 6.3489x; 1.0587x over previous
import jax
import jax.numpy as jnp
from jax.experimental import pallas as pl
from jax.experimental.pallas import tpu as pltpu

E = 64
L = 8
K = 2
NEG = -jnp.inf
OUTR = 8


def _router_kernel(x_ref, w_ref, onb_ref, idx_ref, wgt_ref):
    x = x_ref[...]
    w = w_ref[...]
    dims = (((1,), (1,)), ((), ()))
    sims = jax.lax.dot_general(w, x, dims,
                               preferred_element_type=jnp.float32)

    inv_tn = 1.0 / (jnp.sqrt(jnp.sum(x * x, axis=1, keepdims=True)) + 1e-8)
    inv_en = 1.0 / (jnp.sqrt(jnp.sum(w * w, axis=1, keepdims=True)) + 1e-8)
    cos = jax.lax.dot_general(w * inv_en, x * inv_tn, dims,
                              preferred_element_type=jnp.float32)
    total = cos + 0.1 * onb_ref[...]

    s = sims
    for _ in range(L):
        m = jnp.max(s, axis=0, keepdims=True)
        s = jnp.where(s == m, NEG, s)

    t = jnp.where(s == NEG, total, NEG)
    iota = jax.lax.broadcasted_iota(jnp.int32, t.shape, 0)
    v1 = jnp.max(t, axis=0, keepdims=True)
    i1 = jnp.min(jnp.where(t == v1, iota, E), axis=0, keepdims=True)
    t2 = jnp.where(iota == i1, NEG, t)
    v2 = jnp.max(t2, axis=0, keepdims=True)
    i2 = jnp.min(jnp.where(t2 == v2, iota, E), axis=0, keepdims=True)

    w1 = 1.0 / (1.0 + jnp.exp(v2 - v1))
    T = x.shape[0]
    zi = jnp.zeros((OUTR - K, T), jnp.int32)
    zf = jnp.zeros((OUTR - K, T), jnp.float32)
    idx_ref[...] = jnp.concatenate([i1, i2, zi], axis=0)
    wgt_ref[...] = jnp.concatenate([w1, 1.0 - w1, zf], axis=0)


def kernel(token_embeddings, uzman_embeddings, onbellek_durumu):
    B, S, H = token_embeddings.shape
    N = B * S
    T = 2048
    x = token_embeddings.reshape(N, H)
    onb = onbellek_durumu.reshape(E, 1)

    idx, wgt = pl.pallas_call(
        _router_kernel,
        grid=(N // T,),
        in_specs=[
            pl.BlockSpec((T, H), lambda i: (i, 0)),
            pl.BlockSpec((E, H), lambda i: (0, 0)),
            pl.BlockSpec((E, 1), lambda i: (0, 0)),
        ],
        out_specs=[
            pl.BlockSpec((OUTR, T), lambda i: (0, i)),
            pl.BlockSpec((OUTR, T), lambda i: (0, i)),
        ],
        out_shape=[
            jax.ShapeDtypeStruct((OUTR, N), jnp.int32),
            jax.ShapeDtypeStruct((OUTR, N), jnp.float32),
        ],
        compiler_params=pltpu.CompilerParams(
            dimension_semantics=("arbitrary",),
        ),
    )(x, uzman_embeddings, onb)

    idx = idx[:K].T.reshape(B, S, K)
    wgt = wgt[:K].T.reshape(B, S, K)
    return idx, wgt


# speedup vs baseline: 6.4771x; 1.0202x over previous
import jax
import jax.numpy as jnp
from jax.experimental import pallas as pl
from jax.experimental.pallas import tpu as pltpu

E = 64
L = 8
K = 2
NEG = -jnp.inf
OUTR = 8


def _router_kernel(x_ref, w_ref, onb_ref, idx_ref, wgt_ref):
    x = x_ref[...]
    w = w_ref[...]
    dims = (((1,), (1,)), ((), ()))
    sims = jax.lax.dot_general(w, x, dims,
                               preferred_element_type=jnp.float32)

    inv_tn = 1.0 / (jnp.sqrt(jnp.sum(x * x, axis=1, keepdims=True)) + 1e-8)
    inv_en = 1.0 / (jnp.sqrt(jnp.sum(w * w, axis=1, keepdims=True)) + 1e-8)
    cos = jax.lax.dot_general(w * inv_en, x * inv_tn, dims,
                              preferred_element_type=jnp.float32)
    total = cos + 0.1 * onb_ref[...]

    s = sims
    for _ in range(L):
        m = jnp.max(s, axis=0, keepdims=True)
        s = jnp.where(s == m, NEG, s)

    t = jnp.where(s == NEG, total, NEG)
    iota = jax.lax.broadcasted_iota(jnp.int32, t.shape, 0)
    v1 = jnp.max(t, axis=0, keepdims=True)
    i1 = jnp.min(jnp.where(t == v1, iota, E), axis=0, keepdims=True)
    t2 = jnp.where(iota == i1, NEG, t)
    v2 = jnp.max(t2, axis=0, keepdims=True)
    i2 = jnp.min(jnp.where(t2 == v2, iota, E), axis=0, keepdims=True)

    w1 = 1.0 / (1.0 + jnp.exp(v2 - v1))
    T = x.shape[0]
    zi = jnp.zeros((OUTR - K, T), jnp.int32)
    zf = jnp.zeros((OUTR - K, T), jnp.float32)
    idx_ref[...] = jnp.concatenate([i1, i2, zi], axis=0)
    wgt_ref[...] = jnp.concatenate([w1, 1.0 - w1, zf], axis=0)


def kernel(token_embeddings, uzman_embeddings, onbellek_durumu):
    B, S, H = token_embeddings.shape
    N = B * S
    T = 4096
    x = token_embeddings.reshape(N, H)
    onb = onbellek_durumu.reshape(E, 1)

    idx, wgt = pl.pallas_call(
        _router_kernel,
        grid=(N // T,),
        in_specs=[
            pl.BlockSpec((T, H), lambda i: (i, 0)),
            pl.BlockSpec((E, H), lambda i: (0, 0)),
            pl.BlockSpec((E, 1), lambda i: (0, 0)),
        ],
        out_specs=[
            pl.BlockSpec((OUTR, T), lambda i: (0, i)),
            pl.BlockSpec((OUTR, T), lambda i: (0, i)),
        ],
        out_shape=[
            jax.ShapeDtypeStruct((OUTR, N), jnp.int32),
            jax.ShapeDtypeStruct((OUTR, N), jnp.float32),
        ],
        compiler_params=pltpu.CompilerParams(
            dimension_semantics=("arbitrary",),
        ),
    )(x, uzman_embeddings, onb)

    idx = idx[:K].T.reshape(B, S, K)
    wgt = wgt[:K].T.reshape(B, S, K)
    return idx, wgt
